# Initial kernel scaffold; baseline (speedup 1.0000x reference)
import jax
import jax.numpy as jnp
from jax.experimental import pallas as pl


def kernel(heatmaps, center, cameraMatrices):
    raise NotImplementedError("write your pallas kernel here")


# speedup vs baseline: 9.9357x; 9.9357x over previous
import functools

import jax
import jax.numpy as jnp
from jax import lax
from jax.experimental import pallas as pl
from jax.experimental.pallas import tpu as pltpu
from jax.experimental.pallas import tpu_sc as plsc

C = 12
J = 8
H, W = 512, 640
HW = H * W
GS = 48
N = GS ** 3
SPACING = 2.0
LANES = 128
ROWS = N // LANES

NUM_CORES = 2
NUM_SUBCORES = 16
NW = NUM_CORES * NUM_SUBCORES
CHUNK = N // NW
VL = 16


def _indices_body(center_ref, m_ref, out_ref):
    r = lax.broadcasted_iota(jnp.int32, (ROWS, LANES), 0)
    l = lax.broadcasted_iota(jnp.int32, (ROWS, LANES), 1)
    n = r * LANES + l
    gi = n // (GS * GS)
    gj = (n // GS) % GS
    gk = n % GS

    def wrap(t):
        return jnp.where(t < GS // 2, t, t - GS).astype(jnp.float32) * SPACING

    gx = wrap(gi) + center_ref[0]
    gy = wrap(gj) + center_ref[1]
    gz = wrap(gk) + center_ref[2]
    for c in range(C):
        p0 = gx * m_ref[c, 0, 0] + gy * m_ref[c, 1, 0] + gz * m_ref[c, 2, 0] + m_ref[c, 3, 0]
        p1 = gx * m_ref[c, 0, 1] + gy * m_ref[c, 1, 1] + gz * m_ref[c, 2, 1] + m_ref[c, 3, 1]
        u = jnp.clip(p0 / gz, 0.0, 1279.0)
        v = jnp.clip(p1 / gz, 0.0, 1023.0)
        out_ref[c] = (v * 0.5).astype(jnp.int32) * W + (u * 0.5).astype(jnp.int32)


def _compute_indices(center, cameraMatrices):
    out = pl.pallas_call(
        _indices_body,
        out_shape=jax.ShapeDtypeStruct((C, ROWS, LANES), jnp.int32),
        in_specs=[
            pl.BlockSpec(memory_space=pltpu.SMEM),
            pl.BlockSpec(memory_space=pltpu.SMEM),
        ],
    )(center, cameraMatrices)
    return out.reshape(C, N)


SPAN = 3072
ALIGN = 128


def _hreduce(vec, op):
    m = vec[0]
    for i in range(1, VL):
        m = op(m, vec[i])
    return m


def _gather_body(heat, idx_hbm, out_hbm, idx_v, win, work, acc, semi, semw0, semw1):
    wid = lax.axis_index("sub") * NUM_CORES + lax.axis_index("core")
    base = wid * CHUNK
    descs = [
        pltpu.async_copy(
            idx_hbm.at[c, pl.ds(base, CHUNK)],
            idx_v.at[pl.ds(c * CHUNK, CHUNK)],
            semi,
        )
        for c in range(C)
    ]
    for d in descs:
        d.wait()

    zeros = jnp.zeros((VL,), jnp.float32)
    for j in range(J):
        def zb(t, carry):
            acc[j, pl.ds(t * VL, VL)] = zeros
            return carry
        lax.fori_loop(0, CHUNK // VL, zb, 0)

    los = []
    his = []
    for c in range(C):
        def mmb(t, carry):
            mn, mx = carry
            v = idx_v[pl.ds(c * CHUNK + t * VL, VL)]
            return jnp.minimum(mn, v), jnp.maximum(mx, v)
        mn, mx = lax.fori_loop(
            0, CHUNK // VL, mmb,
            (jnp.full((VL,), HW, jnp.int32), jnp.zeros((VL,), jnp.int32)),
        )
        los.append(_hreduce(mn, jnp.minimum))
        his.append(_hreduce(mx, jnp.maximum))

    def win_start(lo):
        a = jnp.minimum(lo & ~(ALIGN - 1), HW - SPAN)
        return pl.multiple_of(a, ALIGN)

    semw = (semw0, semw1)

    def wslice(slot, j):
        return win.at[pl.ds((slot * J + j) * SPAN, SPAN)]

    def fire_win(c, slot, start):
        return [
            pltpu.async_copy(
                heat.at[c, j, pl.ds(start, SPAN)], wslice(slot, j), semw[slot]
            )
            for j in range(J)
        ]

    wd = [None] * C
    wd[0] = fire_win(0, 0, win_start(los[0]))
    for c in range(C):
        slot = c % 2
        if c + 1 < C:
            wd[c + 1] = fire_win(c + 1, (c + 1) % 2, win_start(los[c + 1]))
        for d in wd[c]:
            d.wait()
        lo_al = win_start(los[c])
        fast = (his[c] - lo_al) < SPAN

        @pl.when(fast)
        def _():
            def fb(t, carry):
                s = pl.ds(t * VL, VL)
                local = idx_v[pl.ds(c * CHUNK + t * VL, VL)] - lo_al
                for j in range(J):
                    g = plsc.load_gather(wslice(slot, j), [local])
                    plsc.addupdate(acc.at[j, s], g)
                return carry
            lax.fori_loop(0, CHUNK // VL, fb, 0)

        @pl.when(jnp.logical_not(fast))
        def _():
            def cb(t, carry):
                s = pl.ds(t * VL, VL)
                work[s] = idx_v[pl.ds(c * CHUNK + t * VL, VL)]
                return carry
            lax.fori_loop(0, CHUNK // VL, cb, 0)

            def cond(lo2):
                return lo2 < HW

            def body(lo2):
                lo2a = win_start(lo2)
                for j in range(J):
                    pltpu.sync_copy(
                        heat.at[c, j, pl.ds(lo2a, SPAN)], wslice(slot, j)
                    )

                def pb(t, carry):
                    s = pl.ds(t * VL, VL)
                    w = work[s]
                    rel = w - lo2a
                    m = rel < SPAN
                    local = jnp.minimum(rel, SPAN - 1)
                    for j in range(J):
                        g = plsc.load_gather(wslice(slot, j), [local])
                        plsc.addupdate(acc.at[j, s], jnp.where(m, g, 0.0))
                    work[s] = jnp.where(m, HW, w)
                    return carry

                lax.fori_loop(0, CHUNK // VL, pb, 0)

                def mmb2(t, carry):
                    return jnp.minimum(carry, work[pl.ds(t * VL, VL)])

                mn = lax.fori_loop(
                    0, CHUNK // VL, mmb2, jnp.full((VL,), HW, jnp.int32)
                )
                return _hreduce(mn, jnp.minimum)

            lax.while_loop(cond, body, los[c])

    scale = jnp.float32(1.0 / C)
    for j in range(J):
        def sb(t, carry):
            s = pl.ds(t * VL, VL)
            acc[j, s] = acc[j, s] * scale
            return carry
        lax.fori_loop(0, CHUNK // VL, sb, 0)
        pltpu.sync_copy(acc.at[j], out_hbm.at[j, pl.ds(base, CHUNK)])


@functools.cache
def _make_gather():
    return functools.partial(
        pl.kernel,
        out_type=jax.ShapeDtypeStruct((J, N), jnp.float32),
        compiler_params=pltpu.CompilerParams(needs_layout_passes=False),
        mesh=plsc.VectorSubcoreMesh(
            core_axis_name="core",
            subcore_axis_name="sub",
            num_cores=NUM_CORES,
            num_subcores=NUM_SUBCORES,
        ),
        scratch_types=[
            pltpu.VMEM((C * CHUNK,), jnp.int32),
            pltpu.VMEM((2 * J * SPAN,), jnp.float32),
            pltpu.VMEM((CHUNK,), jnp.int32),
            pltpu.VMEM((J, CHUNK), jnp.float32),
            pltpu.SemaphoreType.DMA,
            pltpu.SemaphoreType.DMA,
            pltpu.SemaphoreType.DMA,
        ],
    )(_gather_body)


def kernel(heatmaps, center, cameraMatrices):
    b, c, j, h, w = heatmaps.shape
    idx = _compute_indices(center, cameraMatrices)
    heat = heatmaps.reshape(c, j, h * w)
    out = _make_gather()(heat, idx)
    return out.reshape(b, j, GS, GS, GS)


# speedup vs baseline: 12.8704x; 1.2954x over previous
import functools

import jax
import jax.numpy as jnp
from jax import lax
from jax.experimental import pallas as pl
from jax.experimental.pallas import tpu as pltpu
from jax.experimental.pallas import tpu_sc as plsc

C = 12
J = 8
H, W = 512, 640
HW = H * W
GS = 48
N = GS ** 3
SPACING = 2.0
LANES = 128
ROWS = N // LANES

NUM_CORES = 2
NUM_SUBCORES = 16
NW = NUM_CORES * NUM_SUBCORES
CHUNK = N // NW
VL = 16


def _indices_body(center_ref, m_ref, out_ref):
    r = lax.broadcasted_iota(jnp.int32, (ROWS, LANES), 0)
    l = lax.broadcasted_iota(jnp.int32, (ROWS, LANES), 1)
    n = r * LANES + l
    gi = n // (GS * GS)
    gj = (n // GS) % GS
    gk = n % GS

    def wrap(t):
        return jnp.where(t < GS // 2, t, t - GS).astype(jnp.float32) * SPACING

    gx = wrap(gi) + center_ref[0]
    gy = wrap(gj) + center_ref[1]
    gz = wrap(gk) + center_ref[2]
    for c in range(C):
        p0 = gx * m_ref[c, 0, 0] + gy * m_ref[c, 1, 0] + gz * m_ref[c, 2, 0] + m_ref[c, 3, 0]
        p1 = gx * m_ref[c, 0, 1] + gy * m_ref[c, 1, 1] + gz * m_ref[c, 2, 1] + m_ref[c, 3, 1]
        u = jnp.clip(p0 / gz, 0.0, 1279.0)
        v = jnp.clip(p1 / gz, 0.0, 1023.0)
        out_ref[c] = (v * 0.5).astype(jnp.int32) * W + (u * 0.5).astype(jnp.int32)


def _compute_indices(center, cameraMatrices):
    out = pl.pallas_call(
        _indices_body,
        out_shape=jax.ShapeDtypeStruct((C, ROWS, LANES), jnp.int32),
        in_specs=[
            pl.BlockSpec(memory_space=pltpu.SMEM),
            pl.BlockSpec(memory_space=pltpu.SMEM),
        ],
    )(center, cameraMatrices)
    return out.reshape(C, N)


SPAN = 3072
ALIGN = 128


def _hreduce(vec, op):
    m = vec[0]
    for i in range(1, VL):
        m = op(m, vec[i])
    return m


def _gather_body(heat, idx_hbm, out_hbm, idx_v, win, work, acc, semi, semw0, semw1):
    wid = lax.axis_index("sub") * NUM_CORES + lax.axis_index("core")
    base = wid * CHUNK
    descs = [
        pltpu.async_copy(
            idx_hbm.at[c, pl.ds(base, CHUNK)],
            idx_v.at[pl.ds(c * CHUNK, CHUNK)],
            semi,
        )
        for c in range(C)
    ]
    for d in descs:
        d.wait()

    zeros = jnp.zeros((VL,), jnp.float32)
    for j in range(J):
        def zb(t, carry):
            acc[j, pl.ds(t * VL, VL)] = zeros
            return carry
        lax.fori_loop(0, CHUNK // VL, zb, 0)

    los = []
    his = []
    for c in range(C):
        def mmb(t, carry):
            mn, mx = carry
            v = idx_v[pl.ds(c * CHUNK + t * VL, VL)]
            return jnp.minimum(mn, v), jnp.maximum(mx, v)
        mn, mx = lax.fori_loop(
            0, CHUNK // VL, mmb,
            (jnp.full((VL,), HW, jnp.int32), jnp.zeros((VL,), jnp.int32)),
        )
        los.append(_hreduce(mn, jnp.minimum))
        his.append(_hreduce(mx, jnp.maximum))

    def win_start(lo):
        a = jnp.minimum(lo & ~(ALIGN - 1), HW - SPAN)
        return pl.multiple_of(a, ALIGN)

    semw = (semw0, semw1)

    def wslice(slot, j):
        return win.at[pl.ds((slot * J + j) * SPAN, SPAN)]

    def fire_win(c, slot, start):
        return [
            pltpu.async_copy(
                heat.at[c, j, pl.ds(start, SPAN)], wslice(slot, j), semw[slot]
            )
            for j in range(J)
        ]

    wd = [None] * C
    wd[0] = fire_win(0, 0, win_start(los[0]))
    for c in range(C):
        slot = c % 2
        if c + 1 < C:
            wd[c + 1] = fire_win(c + 1, (c + 1) % 2, win_start(los[c + 1]))
        for d in wd[c]:
            d.wait()
        lo_al = win_start(los[c])
        fast = (his[c] - lo_al) < SPAN

        @pl.when(fast)
        def _():
            UNROLL = 4

            def fb(t, carry):
                locs = []
                for u in range(UNROLL):
                    off = c * CHUNK + (t * UNROLL + u) * VL
                    locs.append(idx_v[pl.ds(off, VL)] - lo_al)
                gs = [
                    [plsc.load_gather(wslice(slot, j), [locs[u]]) for j in range(J)]
                    for u in range(UNROLL)
                ]
                for u in range(UNROLL):
                    s = pl.ds((t * UNROLL + u) * VL, VL)
                    for j in range(J):
                        plsc.addupdate(acc.at[j, s], gs[u][j])
                return carry

            lax.fori_loop(0, CHUNK // (VL * UNROLL), fb, 0)

        @pl.when(jnp.logical_not(fast))
        def _():
            def cb(t, carry):
                s = pl.ds(t * VL, VL)
                work[s] = idx_v[pl.ds(c * CHUNK + t * VL, VL)]
                return carry
            lax.fori_loop(0, CHUNK // VL, cb, 0)

            def cond(lo2):
                return lo2 < HW

            def body(lo2):
                lo2a = win_start(lo2)
                for j in range(J):
                    pltpu.sync_copy(
                        heat.at[c, j, pl.ds(lo2a, SPAN)], wslice(slot, j)
                    )

                def pb(t, carry):
                    s = pl.ds(t * VL, VL)
                    w = work[s]
                    rel = w - lo2a
                    m = rel < SPAN
                    local = jnp.minimum(rel, SPAN - 1)
                    for j in range(J):
                        g = plsc.load_gather(wslice(slot, j), [local])
                        plsc.addupdate(acc.at[j, s], jnp.where(m, g, 0.0))
                    work[s] = jnp.where(m, HW, w)
                    return carry

                lax.fori_loop(0, CHUNK // VL, pb, 0)

                def mmb2(t, carry):
                    return jnp.minimum(carry, work[pl.ds(t * VL, VL)])

                mn = lax.fori_loop(
                    0, CHUNK // VL, mmb2, jnp.full((VL,), HW, jnp.int32)
                )
                return _hreduce(mn, jnp.minimum)

            lax.while_loop(cond, body, los[c])

    scale = jnp.float32(1.0 / C)
    for j in range(J):
        def sb(t, carry):
            s = pl.ds(t * VL, VL)
            acc[j, s] = acc[j, s] * scale
            return carry
        lax.fori_loop(0, CHUNK // VL, sb, 0)
        pltpu.sync_copy(acc.at[j], out_hbm.at[j, pl.ds(base, CHUNK)])


@functools.cache
def _make_gather():
    return functools.partial(
        pl.kernel,
        out_type=jax.ShapeDtypeStruct((J, N), jnp.float32),
        compiler_params=pltpu.CompilerParams(needs_layout_passes=False),
        mesh=plsc.VectorSubcoreMesh(
            core_axis_name="core",
            subcore_axis_name="sub",
            num_cores=NUM_CORES,
            num_subcores=NUM_SUBCORES,
        ),
        scratch_types=[
            pltpu.VMEM((C * CHUNK,), jnp.int32),
            pltpu.VMEM((2 * J * SPAN,), jnp.float32),
            pltpu.VMEM((CHUNK,), jnp.int32),
            pltpu.VMEM((J, CHUNK), jnp.float32),
            pltpu.SemaphoreType.DMA,
            pltpu.SemaphoreType.DMA,
            pltpu.SemaphoreType.DMA,
        ],
    )(_gather_body)


def kernel(heatmaps, center, cameraMatrices):
    b, c, j, h, w = heatmaps.shape
    idx = _compute_indices(center, cameraMatrices)
    heat = heatmaps.reshape(c, j, h * w)
    out = _make_gather()(heat, idx)
    return out.reshape(b, j, GS, GS, GS)


# speedup vs baseline: 13.3676x; 1.0386x over previous
import functools

import jax
import jax.numpy as jnp
from jax import lax
from jax.experimental import pallas as pl
from jax.experimental.pallas import tpu as pltpu
from jax.experimental.pallas import tpu_sc as plsc

C = 12
J = 8
H, W = 512, 640
HW = H * W
GS = 48
N = GS ** 3
SPACING = 2.0
LANES = 128
ROWS = N // LANES

NUM_CORES = 2
NUM_SUBCORES = 16
NW = NUM_CORES * NUM_SUBCORES
CHUNK = N // NW
VL = 16


def _indices_body(center_ref, m_ref, out_ref):
    r = lax.broadcasted_iota(jnp.int32, (ROWS, LANES), 0)
    l = lax.broadcasted_iota(jnp.int32, (ROWS, LANES), 1)
    n = r * LANES + l
    gi = n // (GS * GS)
    gj = (n // GS) % GS
    gk = n % GS

    def wrap(t):
        return jnp.where(t < GS // 2, t, t - GS).astype(jnp.float32) * SPACING

    gx = wrap(gi) + center_ref[0]
    gy = wrap(gj) + center_ref[1]
    gz = wrap(gk) + center_ref[2]
    for c in range(C):
        p0 = gx * m_ref[c, 0, 0] + gy * m_ref[c, 1, 0] + gz * m_ref[c, 2, 0] + m_ref[c, 3, 0]
        p1 = gx * m_ref[c, 0, 1] + gy * m_ref[c, 1, 1] + gz * m_ref[c, 2, 1] + m_ref[c, 3, 1]
        u = jnp.clip(p0 / gz, 0.0, 1279.0)
        v = jnp.clip(p1 / gz, 0.0, 1023.0)
        out_ref[c] = (v * 0.5).astype(jnp.int32) * W + (u * 0.5).astype(jnp.int32)


def _compute_indices(center, cameraMatrices):
    out = pl.pallas_call(
        _indices_body,
        out_shape=jax.ShapeDtypeStruct((C, ROWS, LANES), jnp.int32),
        in_specs=[
            pl.BlockSpec(memory_space=pltpu.SMEM),
            pl.BlockSpec(memory_space=pltpu.SMEM),
        ],
    )(center, cameraMatrices)
    return out.reshape(C * N)


SPAN = 3072
ALIGN = 128


def _hreduce(vec, op):
    m = vec[0]
    for i in range(1, VL):
        m = op(m, vec[i])
    return m


def _gather_body(heat, idx_hbm, out_hbm, idx_v, win, work, acc, semi, semw0, semw1):
    wid = lax.axis_index("sub") * NUM_CORES + lax.axis_index("core")
    base = wid * CHUNK
    descs = [
        pltpu.async_copy(
            idx_hbm.at[pl.ds(c * N + base, CHUNK)],
            idx_v.at[pl.ds(c * CHUNK, CHUNK)],
            semi,
        )
        for c in range(C)
    ]
    for d in descs:
        d.wait()

    zeros = jnp.zeros((VL,), jnp.float32)
    for j in range(J):
        def zb(t, carry):
            acc[j, pl.ds(t * VL, VL)] = zeros
            return carry
        lax.fori_loop(0, CHUNK // VL, zb, 0)

    los = []
    his = []
    for c in range(C):
        def mmb(t, carry):
            mn, mx = carry
            v = idx_v[pl.ds(c * CHUNK + t * VL, VL)]
            return jnp.minimum(mn, v), jnp.maximum(mx, v)
        mn, mx = lax.fori_loop(
            0, CHUNK // VL, mmb,
            (jnp.full((VL,), HW, jnp.int32), jnp.zeros((VL,), jnp.int32)),
        )
        los.append(_hreduce(mn, jnp.minimum))
        his.append(_hreduce(mx, jnp.maximum))

    def win_start(lo):
        a = jnp.minimum(lo & ~(ALIGN - 1), HW - SPAN)
        return pl.multiple_of(a, ALIGN)

    semw = (semw0, semw1)

    def wslice(slot, j):
        return win.at[pl.ds((slot * J + j) * SPAN, SPAN)]

    def fire_win(c, slot, start):
        return [
            pltpu.async_copy(
                heat.at[c, j, pl.ds(start, SPAN)], wslice(slot, j), semw[slot]
            )
            for j in range(J)
        ]

    wd = [None] * C
    wd[0] = fire_win(0, 0, win_start(los[0]))
    for c in range(C):
        slot = c % 2
        if c + 1 < C:
            wd[c + 1] = fire_win(c + 1, (c + 1) % 2, win_start(los[c + 1]))
        for d in wd[c]:
            d.wait()
        lo_al = win_start(los[c])
        fast = (his[c] - lo_al) < SPAN

        @pl.when(fast)
        def _():
            UNROLL = 4

            def fb(t, carry):
                locs = []
                for u in range(UNROLL):
                    off = c * CHUNK + (t * UNROLL + u) * VL
                    locs.append(idx_v[pl.ds(off, VL)] - lo_al)
                gs = [
                    [plsc.load_gather(wslice(slot, j), [locs[u]]) for j in range(J)]
                    for u in range(UNROLL)
                ]
                for u in range(UNROLL):
                    s = pl.ds((t * UNROLL + u) * VL, VL)
                    for j in range(J):
                        plsc.addupdate(acc.at[j, s], gs[u][j])
                return carry

            lax.fori_loop(0, CHUNK // (VL * UNROLL), fb, 0)

        @pl.when(jnp.logical_not(fast))
        def _():
            def cb(t, carry):
                s = pl.ds(t * VL, VL)
                work[s] = idx_v[pl.ds(c * CHUNK + t * VL, VL)]
                return carry
            lax.fori_loop(0, CHUNK // VL, cb, 0)

            def cond(lo2):
                return lo2 < HW

            def body(lo2):
                lo2a = win_start(lo2)
                for j in range(J):
                    pltpu.sync_copy(
                        heat.at[c, j, pl.ds(lo2a, SPAN)], wslice(slot, j)
                    )

                def pb(t, carry):
                    s = pl.ds(t * VL, VL)
                    w = work[s]
                    rel = w - lo2a
                    m = rel < SPAN
                    local = jnp.minimum(rel, SPAN - 1)
                    for j in range(J):
                        g = plsc.load_gather(wslice(slot, j), [local])
                        plsc.addupdate(acc.at[j, s], jnp.where(m, g, 0.0))
                    work[s] = jnp.where(m, HW, w)
                    return carry

                lax.fori_loop(0, CHUNK // VL, pb, 0)

                def mmb2(t, carry):
                    return jnp.minimum(carry, work[pl.ds(t * VL, VL)])

                mn = lax.fori_loop(
                    0, CHUNK // VL, mmb2, jnp.full((VL,), HW, jnp.int32)
                )
                return _hreduce(mn, jnp.minimum)

            lax.while_loop(cond, body, los[c])

    scale = jnp.float32(1.0 / C)
    for j in range(J):
        def sb(t, carry):
            s = pl.ds(t * VL, VL)
            acc[j, s] = acc[j, s] * scale
            return carry
        lax.fori_loop(0, CHUNK // VL, sb, 0)
        pltpu.sync_copy(acc.at[j], out_hbm.at[pl.ds(j * N + base, CHUNK)])


@functools.cache
def _make_gather():
    return functools.partial(
        pl.kernel,
        out_type=jax.ShapeDtypeStruct((J * N,), jnp.float32),
        compiler_params=pltpu.CompilerParams(needs_layout_passes=False),
        mesh=plsc.VectorSubcoreMesh(
            core_axis_name="core",
            subcore_axis_name="sub",
            num_cores=NUM_CORES,
            num_subcores=NUM_SUBCORES,
        ),
        scratch_types=[
            pltpu.VMEM((C * CHUNK,), jnp.int32),
            pltpu.VMEM((2 * J * SPAN,), jnp.float32),
            pltpu.VMEM((CHUNK,), jnp.int32),
            pltpu.VMEM((J, CHUNK), jnp.float32),
            pltpu.SemaphoreType.DMA,
            pltpu.SemaphoreType.DMA,
            pltpu.SemaphoreType.DMA,
        ],
    )(_gather_body)


def kernel(heatmaps, center, cameraMatrices):
    b, c, j, h, w = heatmaps.shape
    idx = _compute_indices(center, cameraMatrices)
    heat = heatmaps.reshape(c, j, h * w)
    out = _make_gather()(heat, idx)
    return out.reshape(b, j, GS, GS, GS)
